# baseline (speedup 1.0000x reference)
import jax
import jax.numpy as jnp
from jax.experimental import pallas as pl
from jax.experimental.pallas import tpu as pltpu

NUM_ACTIONS = 128


def _gather_softmax(idx_ref, table_ref, out_ref, row_ref, sem):
    i = idx_ref[0]
    copy = pltpu.make_async_copy(table_ref.at[pl.ds(i, 1)], row_ref, sem)
    copy.start()
    copy.wait()
    row = row_ref[0, :]
    m = jnp.max(row)
    e = jnp.exp(row - m)
    out_ref[:] = e / jnp.sum(e)


@jax.jit
def _policy_table_tc(state_i32, table):
    out = pl.pallas_call(
        _gather_softmax,
        in_specs=[
            pl.BlockSpec(memory_space=pltpu.SMEM),
            pl.BlockSpec(memory_space=pl.ANY),
        ],
        out_specs=pl.BlockSpec(memory_space=pltpu.VMEM),
        out_shape=jax.ShapeDtypeStruct((NUM_ACTIONS,), jnp.float32),
        scratch_shapes=[
            pltpu.VMEM((1, NUM_ACTIONS), jnp.float32),
            pltpu.SemaphoreType.DMA,
        ],
    )(state_i32, table)
    return out


def kernel(state, table):
    state_i32 = state.astype(jnp.int32)
    return _policy_table_tc(state_i32, table)
